# baseline (speedup 1.0000x reference)
import jax
import jax.numpy as jnp
from jax import lax
from jax.experimental import pallas as pl
from jax.experimental.pallas import tpu as pltpu
from jax.experimental.pallas import tpu_sc as plsc

B, R, F = 1024, 200, 128
G, L = 4, 32
OUT = G * L
LANES = 16

NUM_CORES = 2
NUM_SUBCORES = 16
NW = NUM_CORES * NUM_SUBCORES

B_PER_W = B // NW


def _sc_body(x_hbm, gi_hbm, gm_hbm, out_hbm,
             in_v0, in_v1, out_v0, out_v1, idx_v, msk_v,
             si0, si1, so0, so1):
    in_bufs = (in_v0, in_v1)
    out_bufs = (out_v0, out_v1)
    sin = (si0, si1)
    sout = (so0, so1)

    wid = lax.axis_index("s") * NUM_CORES + lax.axis_index("c")
    b0_w = wid * B_PER_W

    pltpu.sync_copy(gi_hbm, idx_v)
    pltpu.sync_copy(gm_hbm, msk_v)

    cols = []
    msks = []
    for j in range(OUT // LANES):
        g, h = j // 2, j % 2
        cj = idx_v[g, pl.ds(h * LANES, LANES)]
        cj = jnp.minimum(jnp.maximum(cj, 0), F - 1)
        cols.append(cj)
        msks.append(msk_v[g, pl.ds(h * LANES, LANES)])

    def in_dma(ci, bf):
        return pltpu.make_async_copy(
            x_hbm.at[b0_w + ci], in_bufs[bf], sin[bf])

    def out_dma(ci, bf):
        return pltpu.make_async_copy(
            out_bufs[bf], out_hbm.at[b0_w + ci], sout[bf])

    in_dma(0, 0).start()
    in_dma(1, 1).start()

    def outer(oi, carry):
        for bf in range(2):
            ci = 2 * oi + bf
            in_dma(ci, bf).wait()

            @pl.when(oi > 0)
            def _():
                out_dma(ci - 2, bf).wait()

            @plsc.parallel_loop(0, R, step=1, unroll=8)
            def row_body(r):
                rbase = jnp.full((LANES,), r * F, dtype=jnp.int32)
                obase = r * OUT
                for j in range(OUT // LANES):
                    v = plsc.load_gather(in_bufs[bf], [rbase + cols[j]])
                    out_bufs[bf][pl.ds(obase + j * LANES, LANES)] = v * msks[j]

            out_dma(ci, bf).start()

            @pl.when(ci + 2 < B_PER_W)
            def _():
                in_dma(ci + 2, bf).start()
        return carry

    lax.fori_loop(0, B_PER_W // 2, outer, 0, unroll=False)
    out_dma(B_PER_W - 2, 0).wait()
    out_dma(B_PER_W - 1, 1).wait()


@jax.jit
def kernel(x_brd, g_idx, g_mask):
    mesh = plsc.VectorSubcoreMesh(
        core_axis_name="c", subcore_axis_name="s",
        num_cores=NUM_CORES, num_subcores=NUM_SUBCORES)
    scratch = [
        pltpu.VMEM((R * F,), jnp.float32),
        pltpu.VMEM((R * F,), jnp.float32),
        pltpu.VMEM((R * OUT,), jnp.float32),
        pltpu.VMEM((R * OUT,), jnp.float32),
        pltpu.VMEM((G, L), jnp.int32),
        pltpu.VMEM((G, L), jnp.float32),
        pltpu.SemaphoreType.DMA,
        pltpu.SemaphoreType.DMA,
        pltpu.SemaphoreType.DMA,
        pltpu.SemaphoreType.DMA,
    ]
    out = pl.kernel(
        _sc_body,
        out_type=jax.ShapeDtypeStruct((B, R * OUT), jnp.float32),
        mesh=mesh,
        compiler_params=pltpu.CompilerParams(needs_layout_passes=False),
        scratch_types=scratch,
    )(x_brd.reshape(B, R * F), g_idx, g_mask)
    return out.reshape(B, R, G, L)


# speedup vs baseline: 1.3815x; 1.3815x over previous
import jax
import jax.numpy as jnp
from jax import lax
from jax.experimental import pallas as pl
from jax.experimental.pallas import tpu as pltpu
from jax.experimental.pallas import tpu_sc as plsc

B, R, F = 1024, 200, 128
G, L = 4, 32
OUT = G * L
LANES = 16

NUM_CORES = 2
NUM_SUBCORES = 16
NW = NUM_CORES * NUM_SUBCORES

NCHUNK = 1
BC = B // NCHUNK
B_PER_W = BC // NW


def _make_body(b_base):
    def _sc_body(x_hbm, gi_hbm, gm_hbm, out_hbm,
                 in_v0, in_v1, out_v0, out_v1, idx_v, msk_v,
                 si0, si1, so0, so1):
        in_bufs = (in_v0, in_v1)
        out_bufs = (out_v0, out_v1)
        sin = (si0, si1)
        sout = (so0, so1)

        wid = lax.axis_index("s") * NUM_CORES + lax.axis_index("c")
        b0_w = wid * B_PER_W

        pltpu.sync_copy(gi_hbm, idx_v)
        pltpu.sync_copy(gm_hbm, msk_v)

        cols = []
        msks = []
        for j in range(OUT // LANES):
            g, h = j // 2, j % 2
            cj = idx_v[g, pl.ds(h * LANES, LANES)]
            cj = jnp.minimum(jnp.maximum(cj, 0), F - 1)
            cols.append(cj)
            msks.append(msk_v[g, pl.ds(h * LANES, LANES)])

        def in_dma(ci, bf):
            return pltpu.make_async_copy(
                x_hbm.at[b_base + b0_w + ci], in_bufs[bf], sin[bf])

        def out_dma(ci, bf):
            return pltpu.make_async_copy(
                out_bufs[bf], out_hbm.at[b0_w + ci], sout[bf])

        in_dma(0, 0).start()
        in_dma(1, 1).start()

        def outer(oi, carry):
            for bf in range(2):
                ci = 2 * oi + bf
                in_dma(ci, bf).wait()

                @pl.when(oi > 0)
                def _():
                    out_dma(ci - 2, bf).wait()

                @plsc.parallel_loop(0, R, step=1, unroll=4)
                def row_body(r):
                    base = jnp.full((LANES,), r, dtype=jnp.int32)
                    for j in range(OUT // LANES):
                        v = plsc.load_gather(in_bufs[bf], [base, cols[j]])
                        out_bufs[bf][r, pl.ds(j * LANES, LANES)] = v * msks[j]

                out_dma(ci, bf).start()

                @pl.when(ci + 2 < B_PER_W)
                def _():
                    in_dma(ci + 2, bf).start()
            return carry

        lax.fori_loop(0, B_PER_W // 2, outer, 0, unroll=False)
        out_dma(B_PER_W - 2, 0).wait()
        out_dma(B_PER_W - 1, 1).wait()

    return _sc_body


@jax.jit
def kernel(x_brd, g_idx, g_mask):
    mesh = plsc.VectorSubcoreMesh(
        core_axis_name="c", subcore_axis_name="s",
        num_cores=NUM_CORES, num_subcores=NUM_SUBCORES)
    scratch = [
        pltpu.VMEM((R, F), jnp.float32),
        pltpu.VMEM((R, F), jnp.float32),
        pltpu.VMEM((R, OUT), jnp.float32),
        pltpu.VMEM((R, OUT), jnp.float32),
        pltpu.VMEM((G, L), jnp.int32),
        pltpu.VMEM((G, L), jnp.float32),
        pltpu.SemaphoreType.DMA,
        pltpu.SemaphoreType.DMA,
        pltpu.SemaphoreType.DMA,
        pltpu.SemaphoreType.DMA,
    ]
    parts = []
    for k in range(NCHUNK):
        part = pl.kernel(
            _make_body(k * BC),
            out_type=jax.ShapeDtypeStruct((BC, R, OUT), jnp.float32),
            mesh=mesh,
            compiler_params=pltpu.CompilerParams(needs_layout_passes=False),
            scratch_types=scratch,
        )(x_brd, g_idx, g_mask)
        parts.append(part.reshape(BC, R, G, L))
    return jnp.concatenate(parts, axis=0)
